# initial kernel scaffold (unmeasured)
import jax
import jax.numpy as jnp
from jax import lax
from jax.experimental import pallas as pl
from jax.experimental.pallas import tpu as pltpu


def kernel(
    t,
):
    def body(*refs):
        pass

    out_shape = jax.ShapeDtypeStruct(..., jnp.float32)
    return pl.pallas_call(body, out_shape=out_shape)(...)


# baseline (device time: 150460 ns/iter reference)
import jax
import jax.numpy as jnp
from jax import lax
from jax.experimental import pallas as pl
from jax.experimental.pallas import tpu as pltpu

N_DEV = 16


def kernel(t):
    m, n = t.shape
    ch = m // N_DEV

    def body(t_ref, out_ref, rs_buf, ag_buf, rs_send, rs_recv, ag_send, ag_recv):
        my = lax.axis_index("i")
        left = jnp.mod(my - 1, N_DEV)
        right = jnp.mod(my + 1, N_DEV)

        barrier_sem = pltpu.get_barrier_semaphore()
        for nbr in (left, right):
            pl.semaphore_signal(
                barrier_sem, inc=1,
                device_id=(nbr,), device_id_type=pl.DeviceIdType.MESH,
            )
        pl.semaphore_wait(barrier_sem, 2)

        def local_chunk(idx):
            return t_ref[pl.ds(idx * ch, ch), :].astype(jnp.bfloat16)

        rs_buf[0, :, :] = local_chunk(my)
        for s in range(N_DEV - 1):
            rdma = pltpu.make_async_remote_copy(
                src_ref=rs_buf.at[s],
                dst_ref=rs_buf.at[s + 1],
                send_sem=rs_send.at[s],
                recv_sem=rs_recv.at[s],
                device_id=(right,),
                device_id_type=pl.DeviceIdType.MESH,
            )
            rdma.start()
            rdma.wait()
            cidx = jnp.mod(my - 1 - s, N_DEV)
            rs_buf[s + 1, :, :] = rs_buf[s + 1, :, :] + local_chunk(cidx)

        s_val = rs_buf[N_DEV - 1, :, :].astype(jnp.float32)
        r = jnp.maximum(s_val, 0.0)
        y = jnp.tanh(s_val) * s_val * s_val + r * r * r
        own_c = jnp.mod(my + 1, N_DEV)
        out_ref[pl.ds(own_c * ch, ch), :] = y
        ag_buf[0, :, :] = y.astype(jnp.bfloat16)

        for s in range(N_DEV - 1):
            rdma = pltpu.make_async_remote_copy(
                src_ref=ag_buf.at[s],
                dst_ref=ag_buf.at[s + 1],
                send_sem=ag_send.at[s],
                recv_sem=ag_recv.at[s],
                device_id=(right,),
                device_id_type=pl.DeviceIdType.MESH,
            )
            rdma.start()
            rdma.wait()
            cidx = jnp.mod(my - s, N_DEV)
            out_ref[pl.ds(cidx * ch, ch), :] = ag_buf[s + 1, :, :].astype(jnp.float32)

    return pl.pallas_call(
        body,
        out_shape=jax.ShapeDtypeStruct((m, n), jnp.float32),
        in_specs=[pl.BlockSpec(memory_space=pltpu.VMEM)],
        out_specs=pl.BlockSpec(memory_space=pltpu.VMEM),
        scratch_shapes=[
            pltpu.VMEM((N_DEV, ch, n), jnp.bfloat16),
            pltpu.VMEM((N_DEV, ch, n), jnp.bfloat16),
            pltpu.SemaphoreType.DMA((N_DEV - 1,)),
            pltpu.SemaphoreType.DMA((N_DEV - 1,)),
            pltpu.SemaphoreType.DMA((N_DEV - 1,)),
            pltpu.SemaphoreType.DMA((N_DEV - 1,)),
        ],
        compiler_params=pltpu.CompilerParams(collective_id=0),
    )(t)


# device time: 91109 ns/iter; 1.6514x vs baseline; 1.6514x over previous
import jax
import jax.numpy as jnp
from jax import lax
from jax.experimental import pallas as pl
from jax.experimental.pallas import tpu as pltpu

N_DEV = 16
P_PLANE = 4
N_Z = 4


def kernel(t):
    m, n = t.shape
    q = m // P_PLANE
    h = q // 2
    c = h // 2
    bf = jnp.bfloat16

    def body(t_ref, out_ref,
             tb, p1_recv, p2a_send, p2a_recv, p2b_send, p2b_recv,
             p4a_send, p4a_recv, p4b_send, p4b_recv, q_send, q_recv,
             p1_ss, p1_rs, p2a_ss, p2a_rs, p2b_ss, p2b_rs,
             p4a_ss, p4a_rs, p4b_ss, p4b_rs, p5_ss, p5_rs):
        my = lax.axis_index("i")
        p = jnp.mod(my, P_PLANE)
        z = my // P_PLANE
        zlow = jnp.mod(z, 2)
        zhi = z // 2

        plane_base = P_PLANE * z
        zx1 = plane_base ^ 4
        zx2 = plane_base ^ 8
        peer_z1 = (my ^ 4)
        peer_z2 = (my ^ 8)
        del plane_base, zx1, zx2

        peers = [P_PLANE * z + jnp.mod(p + d, P_PLANE) for d in (1, 2, 3)]
        peers += [peer_z1, peer_z2]
        barrier_sem = pltpu.get_barrier_semaphore()
        for nbr in peers:
            pl.semaphore_signal(
                barrier_sem, inc=1,
                device_id=(nbr,), device_id_type=pl.DeviceIdType.MESH,
            )
        pl.semaphore_wait(barrier_sem, len(peers))

        quarters = []
        for qi in range(P_PLANE):
            v = t_ref[qi * q:(qi + 1) * q, :].astype(bf)
            quarters.append(v)
            tb[qi, :, :] = v

        p1 = []
        for d in (1, 2, 3):
            tgt_p = jnp.mod(p + d, P_PLANE)
            rdma = pltpu.make_async_remote_copy(
                src_ref=tb.at[tgt_p],
                dst_ref=p1_recv.at[d - 1],
                send_sem=p1_ss.at[d - 1],
                recv_sem=p1_rs.at[d - 1],
                device_id=(P_PLANE * z + tgt_p,),
                device_id_type=pl.DeviceIdType.MESH,
            )
            rdma.start()
            p1.append(rdma)
        for rdma in p1:
            rdma.wait()

        my_q = jnp.zeros((q, n), dtype=bf)
        for qi in range(P_PLANE):
            my_q = jnp.where(p == qi, quarters[qi], my_q)
        r_sum = (my_q.astype(jnp.float32)
                 + p1_recv[0, :, :].astype(jnp.float32)
                 + p1_recv[1, :, :].astype(jnp.float32)
                 + p1_recv[2, :, :].astype(jnp.float32))
        r_val = r_sum.astype(bf)

        keep1 = jnp.where(zlow == 0, r_val[:h, :], r_val[h:, :])
        send1 = jnp.where(zlow == 0, r_val[h:, :], r_val[:h, :])
        p2a_send[:, :] = send1
        rdma = pltpu.make_async_remote_copy(
            src_ref=p2a_send, dst_ref=p2a_recv,
            send_sem=p2a_ss, recv_sem=p2a_rs,
            device_id=(peer_z1,), device_id_type=pl.DeviceIdType.MESH,
        )
        rdma.start()
        rdma.wait()
        a_val = (keep1.astype(jnp.float32)
                 + p2a_recv[:, :].astype(jnp.float32)).astype(bf)

        keep2 = jnp.where(zhi == 0, a_val[:c, :], a_val[c:, :])
        send2 = jnp.where(zhi == 0, a_val[c:, :], a_val[:c, :])
        p2b_send[:, :] = send2
        rdma = pltpu.make_async_remote_copy(
            src_ref=p2b_send, dst_ref=p2b_recv,
            send_sem=p2b_ss, recv_sem=p2b_rs,
            device_id=(peer_z2,), device_id_type=pl.DeviceIdType.MESH,
        )
        rdma.start()
        rdma.wait()
        s_val = (keep2.astype(jnp.float32)
                 + p2b_recv[:, :].astype(jnp.float32))

        relu = jnp.maximum(s_val, 0.0)
        y = jnp.tanh(s_val) * s_val * s_val + relu * relu * relu
        y_bf = y.astype(bf)

        p4a_send[:, :] = y_bf
        rdma = pltpu.make_async_remote_copy(
            src_ref=p4a_send, dst_ref=p4a_recv,
            send_sem=p4a_ss, recv_sem=p4a_rs,
            device_id=(peer_z2,), device_id_type=pl.DeviceIdType.MESH,
        )
        rdma.start()
        rdma.wait()
        got = p4a_recv[:, :]
        c_lo = jnp.where(zhi == 0, y_bf, got)
        c_hi = jnp.where(zhi == 0, got, y_bf)
        p4b_send[:c, :] = c_lo
        p4b_send[c:, :] = c_hi

        rdma = pltpu.make_async_remote_copy(
            src_ref=p4b_send, dst_ref=p4b_recv,
            send_sem=p4b_ss, recv_sem=p4b_rs,
            device_id=(peer_z1,), device_id_type=pl.DeviceIdType.MESH,
        )
        rdma.start()
        rdma.wait()
        blk = p4b_recv[:, :]
        mine = p4b_send[:, :]
        f_lo = jnp.where(zlow == 0, mine, blk)
        f_hi = jnp.where(zlow == 0, blk, mine)
        q_send[:h, :] = f_lo
        q_send[h:, :] = f_hi

        out_ref[pl.ds(p * q, q), :] = q_send[:, :].astype(jnp.float32)

        p5 = []
        for d in (1, 2, 3):
            tgt_p = jnp.mod(p + d, P_PLANE)
            rdma = pltpu.make_async_remote_copy(
                src_ref=q_send,
                dst_ref=q_recv.at[d - 1],
                send_sem=p5_ss.at[d - 1],
                recv_sem=p5_rs.at[d - 1],
                device_id=(P_PLANE * z + tgt_p,),
                device_id_type=pl.DeviceIdType.MESH,
            )
            rdma.start()
            p5.append(rdma)
        for rdma in p5:
            rdma.wait()
        for d in (1, 2, 3):
            src_p = jnp.mod(p - d, P_PLANE)
            out_ref[pl.ds(src_p * q, q), :] = (
                q_recv[d - 1, :, :].astype(jnp.float32))

    return pl.pallas_call(
        body,
        out_shape=jax.ShapeDtypeStruct((m, n), jnp.float32),
        in_specs=[pl.BlockSpec(memory_space=pltpu.VMEM)],
        out_specs=pl.BlockSpec(memory_space=pltpu.VMEM),
        scratch_shapes=[
            pltpu.VMEM((P_PLANE, q, n), bf),
            pltpu.VMEM((3, q, n), bf),
            pltpu.VMEM((h, n), bf),
            pltpu.VMEM((h, n), bf),
            pltpu.VMEM((c, n), bf),
            pltpu.VMEM((c, n), bf),
            pltpu.VMEM((c, n), bf),
            pltpu.VMEM((c, n), bf),
            pltpu.VMEM((h, n), bf),
            pltpu.VMEM((h, n), bf),
            pltpu.VMEM((q, n), bf),
            pltpu.VMEM((3, q, n), bf),
            pltpu.SemaphoreType.DMA((3,)),
            pltpu.SemaphoreType.DMA((3,)),
            pltpu.SemaphoreType.DMA,
            pltpu.SemaphoreType.DMA,
            pltpu.SemaphoreType.DMA,
            pltpu.SemaphoreType.DMA,
            pltpu.SemaphoreType.DMA,
            pltpu.SemaphoreType.DMA,
            pltpu.SemaphoreType.DMA,
            pltpu.SemaphoreType.DMA,
            pltpu.SemaphoreType.DMA((3,)),
            pltpu.SemaphoreType.DMA((3,)),
        ],
        compiler_params=pltpu.CompilerParams(collective_id=0),
    )(t)


# device time: 80324 ns/iter; 1.8732x vs baseline; 1.1343x over previous
import jax
import jax.numpy as jnp
from jax import lax
from jax.experimental import pallas as pl
from jax.experimental.pallas import tpu as pltpu

N_DEV = 16
P_PLANE = 4
N_Z = 4
N_H = 2


def kernel(t):
    m, n = t.shape
    q = m // P_PLANE
    h = q // 2
    c = h // 2
    w = n // N_H
    bf = jnp.bfloat16

    def body(t_ref, out_ref,
             tb, p1_recv, p2a_send, p2a_recv, p2b_send, p2b_recv,
             p4a_send, p4a_recv, p4b_send, p4b_recv, q_send, q_recv,
             p1_ss, p1_rs, p2a_ss, p2a_rs, p2b_ss, p2b_rs,
             p4a_ss, p4a_rs, p4b_ss, p4b_rs, p5_ss, p5_rs):
        my = lax.axis_index("i")
        p = jnp.mod(my, P_PLANE)
        z = my // P_PLANE
        zlow = jnp.mod(z, 2)
        zhi = z // 2
        peer_z1 = my ^ 4
        peer_z2 = my ^ 8

        peers = [P_PLANE * z + jnp.mod(p + d, P_PLANE) for d in (1, 2, 3)]
        peers += [peer_z1, peer_z2]
        barrier_sem = pltpu.get_barrier_semaphore()
        for nbr in peers:
            pl.semaphore_signal(
                barrier_sem, inc=1,
                device_id=(nbr,), device_id_type=pl.DeviceIdType.MESH,
            )
        pl.semaphore_wait(barrier_sem, len(peers))

        cols = [slice(0, w), slice(w, n)]


        def p1_start(hi):
            quarters = []
            for qi in range(P_PLANE):
                v = t_ref[qi * q:(qi + 1) * q, cols[hi]].astype(bf)
                quarters.append(v)
                tb[hi, qi, :, :] = v
            rdmas = []
            for d in (1, 2, 3):
                tgt_p = jnp.mod(p + d, P_PLANE)
                rdma = pltpu.make_async_remote_copy(
                    src_ref=tb.at[hi].at[tgt_p],
                    dst_ref=p1_recv.at[hi].at[d - 1],
                    send_sem=p1_ss.at[hi, d - 1],
                    recv_sem=p1_rs.at[hi, d - 1],
                    device_id=(P_PLANE * z + tgt_p,),
                    device_id_type=pl.DeviceIdType.MESH,
                )
                rdma.start()
                rdmas.append(rdma)
            return rdmas, quarters

        def p1_finish(hi, rdmas, quarters):
            for rdma in rdmas:
                rdma.wait()
            my_q = jnp.zeros((q, w), dtype=bf)
            for qi in range(P_PLANE):
                my_q = jnp.where(p == qi, quarters[qi], my_q)
            r32 = (my_q.astype(jnp.float32)
                   + p1_recv[hi, 0, :, :].astype(jnp.float32)
                   + p1_recv[hi, 1, :, :].astype(jnp.float32)
                   + p1_recv[hi, 2, :, :].astype(jnp.float32))
            return r32.astype(bf)

        def p2s1_start(hi, r_val):
            keep1 = jnp.where(zlow == 0, r_val[:h, :], r_val[h:, :])
            p2a_send[hi, :, :] = jnp.where(zlow == 0, r_val[h:, :], r_val[:h, :])
            rdma = pltpu.make_async_remote_copy(
                src_ref=p2a_send.at[hi], dst_ref=p2a_recv.at[hi],
                send_sem=p2a_ss.at[hi], recv_sem=p2a_rs.at[hi],
                device_id=(peer_z1,), device_id_type=pl.DeviceIdType.MESH,
            )
            rdma.start()
            return rdma, keep1

        def p2s2_start(hi, rdma1, keep1):
            rdma1.wait()
            a_val = (keep1.astype(jnp.float32)
                     + p2a_recv[hi, :, :].astype(jnp.float32)).astype(bf)
            keep2 = jnp.where(zhi == 0, a_val[:c, :], a_val[c:, :])
            p2b_send[hi, :, :] = jnp.where(zhi == 0, a_val[c:, :], a_val[:c, :])
            rdma = pltpu.make_async_remote_copy(
                src_ref=p2b_send.at[hi], dst_ref=p2b_recv.at[hi],
                send_sem=p2b_ss.at[hi], recv_sem=p2b_rs.at[hi],
                device_id=(peer_z2,), device_id_type=pl.DeviceIdType.MESH,
            )
            rdma.start()
            return rdma, keep2

        def p4s1_start(hi, rdma2, keep2):
            rdma2.wait()
            s_val = (keep2.astype(jnp.float32)
                     + p2b_recv[hi, :, :].astype(jnp.float32))
            relu = jnp.maximum(s_val, 0.0)
            y = jnp.tanh(s_val) * s_val * s_val + relu * relu * relu
            p4a_send[hi, :, :] = y.astype(bf)
            rdma = pltpu.make_async_remote_copy(
                src_ref=p4a_send.at[hi], dst_ref=p4a_recv.at[hi],
                send_sem=p4a_ss.at[hi], recv_sem=p4a_rs.at[hi],
                device_id=(peer_z2,), device_id_type=pl.DeviceIdType.MESH,
            )
            rdma.start()
            return rdma

        def p4s2_start(hi, rdma):
            rdma.wait()
            y_bf = p4a_send[hi, :, :]
            got = p4a_recv[hi, :, :]
            p4b_send[hi, :c, :] = jnp.where(zhi == 0, y_bf, got)
            p4b_send[hi, c:, :] = jnp.where(zhi == 0, got, y_bf)
            rdma = pltpu.make_async_remote_copy(
                src_ref=p4b_send.at[hi], dst_ref=p4b_recv.at[hi],
                send_sem=p4b_ss.at[hi], recv_sem=p4b_rs.at[hi],
                device_id=(peer_z1,), device_id_type=pl.DeviceIdType.MESH,
            )
            rdma.start()
            return rdma

        def p5_start(hi, rdma):
            rdma.wait()
            blk = p4b_recv[hi, :, :]
            mine = p4b_send[hi, :, :]
            q_send[hi, :h, :] = jnp.where(zlow == 0, mine, blk)
            q_send[hi, h:, :] = jnp.where(zlow == 0, blk, mine)
            out_ref[pl.ds(p * q, q), cols[hi]] = (
                q_send[hi, :, :].astype(jnp.float32))
            rdmas = []
            for d in (1, 2, 3):
                tgt_p = jnp.mod(p + d, P_PLANE)
                rdma = pltpu.make_async_remote_copy(
                    src_ref=q_send.at[hi],
                    dst_ref=q_recv.at[hi].at[d - 1],
                    send_sem=p5_ss.at[hi, d - 1],
                    recv_sem=p5_rs.at[hi, d - 1],
                    device_id=(P_PLANE * z + tgt_p,),
                    device_id_type=pl.DeviceIdType.MESH,
                )
                rdma.start()
                rdmas.append(rdma)
            return rdmas

        def p5_finish(hi, rdmas):
            for rdma in rdmas:
                rdma.wait()
            for d in (1, 2, 3):
                src_p = jnp.mod(p - d, P_PLANE)
                out_ref[pl.ds(src_p * q, q), cols[hi]] = (
                    q_recv[hi, d - 1, :, :].astype(jnp.float32))

        p1a, qa = p1_start(0)
        p1b, qb = p1_start(1)
        r_a = p1_finish(0, p1a, qa)
        p2a_a, keep1_a = p2s1_start(0, r_a)
        r_b = p1_finish(1, p1b, qb)
        p2b_a, keep2_a = p2s2_start(0, p2a_a, keep1_a)
        p2a_b, keep1_b = p2s1_start(1, r_b)
        p4a_a = p4s1_start(0, p2b_a, keep2_a)
        p2b_b, keep2_b = p2s2_start(1, p2a_b, keep1_b)
        p4b_a = p4s2_start(0, p4a_a)
        p4a_b = p4s1_start(1, p2b_b, keep2_b)
        p5a = p5_start(0, p4b_a)
        p4b_b = p4s2_start(1, p4a_b)
        p5_finish(0, p5a)
        p5b = p5_start(1, p4b_b)
        p5_finish(1, p5b)

    return pl.pallas_call(
        body,
        out_shape=jax.ShapeDtypeStruct((m, n), jnp.float32),
        in_specs=[pl.BlockSpec(memory_space=pltpu.VMEM)],
        out_specs=pl.BlockSpec(memory_space=pltpu.VMEM),
        scratch_shapes=[
            pltpu.VMEM((N_H, P_PLANE, q, w), bf),
            pltpu.VMEM((N_H, 3, q, w), bf),
            pltpu.VMEM((N_H, h, w), bf),
            pltpu.VMEM((N_H, h, w), bf),
            pltpu.VMEM((N_H, c, w), bf),
            pltpu.VMEM((N_H, c, w), bf),
            pltpu.VMEM((N_H, c, w), bf),
            pltpu.VMEM((N_H, c, w), bf),
            pltpu.VMEM((N_H, h, w), bf),
            pltpu.VMEM((N_H, h, w), bf),
            pltpu.VMEM((N_H, q, w), bf),
            pltpu.VMEM((N_H, 3, q, w), bf),
            pltpu.SemaphoreType.DMA((N_H, 3)),
            pltpu.SemaphoreType.DMA((N_H, 3)),
            pltpu.SemaphoreType.DMA((N_H,)),
            pltpu.SemaphoreType.DMA((N_H,)),
            pltpu.SemaphoreType.DMA((N_H,)),
            pltpu.SemaphoreType.DMA((N_H,)),
            pltpu.SemaphoreType.DMA((N_H,)),
            pltpu.SemaphoreType.DMA((N_H,)),
            pltpu.SemaphoreType.DMA((N_H,)),
            pltpu.SemaphoreType.DMA((N_H,)),
            pltpu.SemaphoreType.DMA((N_H, 3)),
            pltpu.SemaphoreType.DMA((N_H, 3)),
        ],
        compiler_params=pltpu.CompilerParams(collective_id=0),
    )(t)


# device time: 74039 ns/iter; 2.0322x vs baseline; 1.0849x over previous
import jax
import jax.numpy as jnp
from jax import lax
from jax.experimental import pallas as pl
from jax.experimental.pallas import tpu as pltpu

N_DEV = 16
P_PLANE = 4
N_Z = 4
N_H = 2


def kernel(t):
    m, n = t.shape
    q = m // P_PLANE
    c = q // N_Z
    w = n // N_H
    bf = jnp.bfloat16

    def body(t_ref, out_ref,
             tb, p1_recv, r_stage, z2_recv, y_stage, z4_recv,
             q_send, q_recv,
             p1_ss, p1_rs, z2_ss, z2_rs, z4_ss, z4_rs, p5_ss, p5_rs):
        my = lax.axis_index("i")
        p = jnp.mod(my, P_PLANE)
        z = my // P_PLANE

        peers = [P_PLANE * z + jnp.mod(p + d, P_PLANE) for d in (1, 2, 3)]
        peers += [P_PLANE * jnp.mod(z + d, N_Z) + p for d in (1, 2, 3)]
        barrier_sem = pltpu.get_barrier_semaphore()
        for nbr in peers:
            pl.semaphore_signal(
                barrier_sem, inc=1,
                device_id=(nbr,), device_id_type=pl.DeviceIdType.MESH,
            )
        pl.semaphore_wait(barrier_sem, len(peers))

        cols = [slice(0, w), slice(w, n)]


        def p1_start(hi):
            quarters = []
            for qi in range(P_PLANE):
                v = t_ref[qi * q:(qi + 1) * q, cols[hi]].astype(bf)
                quarters.append(v)
                tb[hi, qi, :, :] = v
            rdmas = []
            for d in (1, 2, 3):
                tgt_p = jnp.mod(p + d, P_PLANE)
                rdma = pltpu.make_async_remote_copy(
                    src_ref=tb.at[hi].at[tgt_p],
                    dst_ref=p1_recv.at[hi].at[d - 1],
                    send_sem=p1_ss.at[hi, d - 1],
                    recv_sem=p1_rs.at[hi, d - 1],
                    device_id=(P_PLANE * z + tgt_p,),
                    device_id_type=pl.DeviceIdType.MESH,
                )
                rdma.start()
                rdmas.append(rdma)
            return rdmas, quarters

        def p2_start(hi, rdmas, quarters):
            for rdma in rdmas:
                rdma.wait()
            my_q = jnp.zeros((q, w), dtype=bf)
            for qi in range(P_PLANE):
                my_q = jnp.where(p == qi, quarters[qi], my_q)
            r32 = (my_q.astype(jnp.float32)
                   + p1_recv[hi, 0, :, :].astype(jnp.float32)
                   + p1_recv[hi, 1, :, :].astype(jnp.float32)
                   + p1_recv[hi, 2, :, :].astype(jnp.float32))
            r_val = r32.astype(bf)
            for s in range(N_Z):
                r_stage[hi, s, :, :] = r_val[s * c:(s + 1) * c, :]
            rdmas = []
            for d in (1, 2, 3):
                tgt_z = jnp.mod(z + d, N_Z)
                rdma = pltpu.make_async_remote_copy(
                    src_ref=r_stage.at[hi].at[tgt_z],
                    dst_ref=z2_recv.at[hi].at[d - 1],
                    send_sem=z2_ss.at[hi, d - 1],
                    recv_sem=z2_rs.at[hi, d - 1],
                    device_id=(P_PLANE * tgt_z + p,),
                    device_id_type=pl.DeviceIdType.MESH,
                )
                rdma.start()
                rdmas.append(rdma)
            own = jnp.zeros((c, w), dtype=bf)
            for s in range(N_Z):
                own = jnp.where(z == s, r_val[s * c:(s + 1) * c, :], own)
            return rdmas, own

        def p4_start(hi, rdmas, own):
            for rdma in rdmas:
                rdma.wait()
            s_val = (own.astype(jnp.float32)
                     + z2_recv[hi, 0, :, :].astype(jnp.float32)
                     + z2_recv[hi, 1, :, :].astype(jnp.float32)
                     + z2_recv[hi, 2, :, :].astype(jnp.float32))
            relu = jnp.maximum(s_val, 0.0)
            y = jnp.tanh(s_val) * s_val * s_val + relu * relu * relu
            y_bf = y.astype(bf)
            y_stage[hi, :, :] = y_bf
            rdmas = []
            for d in (1, 2, 3):
                tgt_z = jnp.mod(z + d, N_Z)
                rdma = pltpu.make_async_remote_copy(
                    src_ref=y_stage.at[hi],
                    dst_ref=z4_recv.at[hi].at[d - 1],
                    send_sem=z4_ss.at[hi, d - 1],
                    recv_sem=z4_rs.at[hi, d - 1],
                    device_id=(P_PLANE * tgt_z + p,),
                    device_id_type=pl.DeviceIdType.MESH,
                )
                rdma.start()
                rdmas.append(rdma)
            return rdmas, y_bf

        def p5_start(hi, rdmas, y_bf):
            for rdma in rdmas:
                rdma.wait()
            q_send[hi, pl.ds(z * c, c), :] = y_bf
            for d in (1, 2, 3):
                src_z = jnp.mod(z - d, N_Z)
                q_send[hi, pl.ds(src_z * c, c), :] = z4_recv[hi, d - 1, :, :]
            rdmas = []
            for d in (1, 2, 3):
                tgt_p = jnp.mod(p + d, P_PLANE)
                rdma = pltpu.make_async_remote_copy(
                    src_ref=q_send.at[hi],
                    dst_ref=q_recv.at[hi].at[d - 1],
                    send_sem=p5_ss.at[hi, d - 1],
                    recv_sem=p5_rs.at[hi, d - 1],
                    device_id=(P_PLANE * z + tgt_p,),
                    device_id_type=pl.DeviceIdType.MESH,
                )
                rdma.start()
                rdmas.append(rdma)
            out_ref[pl.ds(p * q, q), cols[hi]] = (
                q_send[hi, :, :].astype(jnp.float32))
            return rdmas

        def p5_finish(hi, rdmas):
            for rdma in rdmas:
                rdma.wait()
            for d in (1, 2, 3):
                src_p = jnp.mod(p - d, P_PLANE)
                out_ref[pl.ds(src_p * q, q), cols[hi]] = (
                    q_recv[hi, d - 1, :, :].astype(jnp.float32))

        p1a, qa = p1_start(0)
        p1b, qb = p1_start(1)
        p2a, own_a = p2_start(0, p1a, qa)
        p2b, own_b = p2_start(1, p1b, qb)
        p4a, y_a = p4_start(0, p2a, own_a)
        p4b, y_b = p4_start(1, p2b, own_b)
        p5a = p5_start(0, p4a, y_a)
        p5b = p5_start(1, p4b, y_b)
        p5_finish(0, p5a)
        p5_finish(1, p5b)

    return pl.pallas_call(
        body,
        out_shape=jax.ShapeDtypeStruct((m, n), jnp.float32),
        in_specs=[pl.BlockSpec(memory_space=pltpu.VMEM)],
        out_specs=pl.BlockSpec(memory_space=pltpu.VMEM),
        scratch_shapes=[
            pltpu.VMEM((N_H, P_PLANE, q, w), bf),
            pltpu.VMEM((N_H, 3, q, w), bf),
            pltpu.VMEM((N_H, N_Z, c, w), bf),
            pltpu.VMEM((N_H, 3, c, w), bf),
            pltpu.VMEM((N_H, c, w), bf),
            pltpu.VMEM((N_H, 3, c, w), bf),
            pltpu.VMEM((N_H, q, w), bf),
            pltpu.VMEM((N_H, 3, q, w), bf),
            pltpu.SemaphoreType.DMA((N_H, 3)),
            pltpu.SemaphoreType.DMA((N_H, 3)),
            pltpu.SemaphoreType.DMA((N_H, 3)),
            pltpu.SemaphoreType.DMA((N_H, 3)),
            pltpu.SemaphoreType.DMA((N_H, 3)),
            pltpu.SemaphoreType.DMA((N_H, 3)),
            pltpu.SemaphoreType.DMA((N_H, 3)),
            pltpu.SemaphoreType.DMA((N_H, 3)),
        ],
        compiler_params=pltpu.CompilerParams(collective_id=0),
    )(t)
